# baseline (speedup 1.0000x reference)
import functools
import math

import jax
import jax.numpy as jnp
from jax import lax
from jax.experimental import pallas as pl
from jax.experimental.pallas import tpu as pltpu
from jax.experimental.pallas import tpu_sc as plsc

_LANES = 16
_IDX_CHUNK = 128
_NBUF = 4


def _make_sc_gather(b: int, s: int, v: int, d: int, scale: float,
                    num_cores: int, num_subcores: int):
    nw = num_cores * num_subcores
    rows_per_w = b // nw
    groups = rows_per_w // _NBUF
    splits = []
    off = 0
    while off < s:
        width = min(_IDX_CHUNK, s - off)
        splits.append((off, width))
        off += width
    mesh = plsc.VectorSubcoreMesh(core_axis_name="c", subcore_axis_name="s")

    @functools.partial(
        pl.kernel,
        out_type=jax.ShapeDtypeStruct((b, s, d), jnp.float32),
        mesh=mesh,
        scratch_types=[
            pltpu.VMEM((rows_per_w, s), jnp.int32),
            pltpu.VMEM((_NBUF, s, d), jnp.float32),
            pltpu.SemaphoreType.DMA((_NBUF,)),
            pltpu.SemaphoreType.DMA((_NBUF,)),
        ],
        compiler_params=pltpu.CompilerParams(use_tc_tiling_on_sc=False),
    )
    def sc_gather(tok_hbm, table_hbm, out_hbm, idx_v, rows_v, gsem, ssem):
        wid = lax.axis_index("s") * num_cores + lax.axis_index("c")
        base = wid * rows_per_w
        pltpu.sync_copy(tok_hbm.at[pl.ds(base, rows_per_w)], idx_v)

        def issue_gathers(i, buf):
            for off, width in splits:
                pltpu.async_copy(
                    table_hbm.at[idx_v.at[i, pl.ds(off, width)]],
                    rows_v.at[buf, pl.ds(off, width)],
                    gsem.at[buf],
                )

        def wait_gathers(buf):
            pltpu.make_async_copy(
                out_hbm.at[0], rows_v.at[buf], gsem.at[buf]
            ).wait()

        def wait_store(buf):
            pltpu.make_async_copy(
                rows_v.at[buf], out_hbm.at[0], ssem.at[buf]
            ).wait()

        for buf in range(_NBUF):
            issue_gathers(buf, buf)

        def group_body(g, carry):
            i0 = g * _NBUF
            for buf in range(_NBUF):
                i = i0 + buf
                bprev = (buf - 1) % _NBUF

                @pl.when(jnp.logical_and(i >= 1, i + _NBUF - 1 < rows_per_w))
                def _():
                    wait_store(bprev)
                    issue_gathers(i + _NBUF - 1, bprev)

                wait_gathers(buf)

                @plsc.parallel_loop(0, s, unroll=4)
                def _(r):
                    for col in range(d // _LANES):
                        sl = pl.ds(col * _LANES, _LANES)
                        rows_v[buf, r, sl] = rows_v[buf, r, sl] * scale

                pltpu.async_copy(rows_v.at[buf], out_hbm.at[base + i],
                                 ssem.at[buf])
            return carry

        lax.fori_loop(0, groups, group_body, 0)
        for buf in range(_NBUF):
            wait_store(buf)

    return sc_gather


def kernel(tokens, table):
    b, s = tokens.shape
    v, d = table.shape
    info = plsc.get_sparse_core_info()
    nw = info.num_cores * info.num_subcores
    assert b % (nw * _NBUF) == 0 and d % _LANES == 0
    return _make_sc_gather(b, s, v, d, math.sqrt(d), info.num_cores,
                           info.num_subcores)(tokens.astype(jnp.int32), table)
